# baseline (speedup 1.0000x reference)
import jax
import jax.numpy as jnp
from jax import lax
from jax.experimental import pallas as pl
from jax.experimental.pallas import tpu as pltpu
from jax.experimental.pallas import tpu_sc as plsc

SIGMA = 0.2
T = 4096
ROWS = 64 * 128
NC = 2
NS = 16
NW = NC * NS
ROWS_PER_W = ROWS // NW
R = 2
N_CHUNK = ROWS_PER_W // R
N_GRP = T // 16
IN_DEPTH = 8
OUT_DEPTH = 4


def _body(
    x_hbm, idx_hbm, out_hbm,
    idx_v, i0, i1, i2, i3, i4, i5, i6, i7, o0, o1, o2, o3,
    is0, is1, is2, is3, is4, is5, is6, is7, os0, os1, os2, os3,
):
    c = lax.axis_index("c")
    s = lax.axis_index("s")
    wid = s * NC + c
    base = wid * ROWS_PER_W

    ibufs = ((i0, is0), (i1, is1), (i2, is2), (i3, is3),
             (i4, is4), (i5, is5), (i6, is6), (i7, is7))
    obufs = ((o0, os0), (o1, os1), (o2, os2), (o3, os3))

    for j, (in_v, isem) in enumerate(ibufs):
        pltpu.async_copy(x_hbm.at[pl.ds(base + j * R, R)], in_v, isem)
    pltpu.sync_copy(idx_hbm, idx_v)

    @pl.loop(0, N_CHUNK, step=IN_DEPTH)
    def chunk(i):
        for k, (in_v, isem) in enumerate(ibufs):
            ci = i + k
            rb = base + ci * R
            out_v, osem = obufs[k % OUT_DEPTH]

            pltpu.make_async_copy(x_hbm.at[pl.ds(rb, R)], in_v, isem).wait()

            @pl.when(ci >= OUT_DEPTH)
            def _():
                pltpu.make_async_copy(
                    out_v, out_hbm.at[pl.ds(rb - OUT_DEPTH * R, R)], osem
                ).wait()

            @plsc.parallel_loop(0, N_GRP, 1, unroll=4)
            def grp(g):
                iv = idx_v[pl.ds(g * 16, 16)]
                for r in range(R):
                    rv = jnp.full((16,), r, dtype=jnp.int32)
                    vals = plsc.load_gather(in_v, [rv, iv])
                    out_v[r, pl.ds(g * 16, 16)] = vals

            pltpu.async_copy(out_v, out_hbm.at[pl.ds(rb, R)], osem)

            nxt = ci + IN_DEPTH

            @pl.when(nxt < N_CHUNK)
            def _():
                pltpu.async_copy(
                    x_hbm.at[pl.ds(base + nxt * R, R)], in_v, isem
                )

    last = base + (N_CHUNK - OUT_DEPTH) * R
    for k, (out_v, osem) in enumerate(obufs):
        pltpu.make_async_copy(
            out_v, out_hbm.at[pl.ds(last + k * R, R)], osem
        ).wait()


def _make_kernel(interpret=False):
    mesh = plsc.VectorSubcoreMesh(
        core_axis_name="c", subcore_axis_name="s", num_cores=NC, num_subcores=NS
    )
    return pl.kernel(
        _body,
        out_type=jax.ShapeDtypeStruct((ROWS, T), jnp.float32),
        mesh=mesh,
        scratch_types=(
            [pltpu.VMEM((T,), jnp.int32)]
            + [pltpu.VMEM((R, T), jnp.float32)] * (IN_DEPTH + OUT_DEPTH)
            + [pltpu.SemaphoreType.DMA] * (IN_DEPTH + OUT_DEPTH)
        ),
        interpret=interpret,
        compiler_params=pltpu.CompilerParams(needs_layout_passes=False),
    )


def _warp_indices():
    wkey = jax.random.fold_in(jax.random.key(0), 1)
    warp = jnp.cumsum(jax.random.normal(wkey, (T,), dtype=jnp.float32) * SIGMA)
    warp = (warp - warp.min()) / (warp.max() - warp.min()) * (T - 1)
    return jnp.clip(warp.astype(jnp.int32), 0, T - 1)


@jax.jit
def kernel(x):
    idx = _warp_indices()
    out = _make_kernel()(x.reshape(ROWS, T), idx)
    return out.reshape(x.shape)


# speedup vs baseline: 1.0099x; 1.0099x over previous
import jax
import jax.numpy as jnp
from jax import lax
from jax.experimental import pallas as pl
from jax.experimental.pallas import tpu as pltpu
from jax.experimental.pallas import tpu_sc as plsc

SIGMA = 0.2
T = 4096
ROWS = 64 * 128
NC = 2
NS = 16
NW = NC * NS
ROWS_PER_W = ROWS // NW
R = 4
N_CHUNK = ROWS_PER_W // R
N_GRP = T // 16
IN_DEPTH = 4
OUT_DEPTH = 2


def _body(
    x_hbm, idx_hbm, out_hbm,
    idx_v, i0, i1, i2, i3, o0, o1,
    is0, is1, is2, is3, os0, os1,
):
    c = lax.axis_index("c")
    s = lax.axis_index("s")
    wid = s * NC + c

    def row_of(ci):
        return (ci * NW + wid) * R

    ibufs = ((i0, is0), (i1, is1), (i2, is2), (i3, is3))
    obufs = ((o0, os0), (o1, os1))

    for j, (in_v, isem) in enumerate(ibufs):
        pltpu.async_copy(x_hbm.at[pl.ds(row_of(j), R)], in_v, isem)
    pltpu.sync_copy(idx_hbm, idx_v)

    @pl.loop(0, N_CHUNK, step=IN_DEPTH)
    def chunk(i):
        for k, (in_v, isem) in enumerate(ibufs):
            ci = i + k
            rb = row_of(ci)
            out_v, osem = obufs[k % OUT_DEPTH]

            pltpu.make_async_copy(x_hbm.at[pl.ds(rb, R)], in_v, isem).wait()

            @pl.when(ci >= OUT_DEPTH)
            def _():
                pltpu.make_async_copy(
                    out_v, out_hbm.at[pl.ds(row_of(ci - OUT_DEPTH), R)], osem
                ).wait()

            @plsc.parallel_loop(0, N_GRP, 1, unroll=4)
            def grp(g):
                iv = idx_v[pl.ds(g * 16, 16)]
                for r in range(R):
                    rv = jnp.full((16,), r, dtype=jnp.int32)
                    vals = plsc.load_gather(in_v, [rv, iv])
                    out_v[r, pl.ds(g * 16, 16)] = vals

            pltpu.async_copy(out_v, out_hbm.at[pl.ds(rb, R)], osem)

            nxt = ci + IN_DEPTH

            @pl.when(nxt < N_CHUNK)
            def _():
                pltpu.async_copy(
                    x_hbm.at[pl.ds(row_of(nxt), R)], in_v, isem
                )

    pltpu.make_async_copy(
        o0, out_hbm.at[pl.ds(row_of(N_CHUNK - 2), R)], os0
    ).wait()
    pltpu.make_async_copy(
        o1, out_hbm.at[pl.ds(row_of(N_CHUNK - 1), R)], os1
    ).wait()


def _make_kernel(interpret=False):
    mesh = plsc.VectorSubcoreMesh(
        core_axis_name="c", subcore_axis_name="s", num_cores=NC, num_subcores=NS
    )
    return pl.kernel(
        _body,
        out_type=jax.ShapeDtypeStruct((ROWS, T), jnp.float32),
        mesh=mesh,
        scratch_types=[
            pltpu.VMEM((T,), jnp.int32),
            pltpu.VMEM((R, T), jnp.float32),
            pltpu.VMEM((R, T), jnp.float32),
            pltpu.VMEM((R, T), jnp.float32),
            pltpu.VMEM((R, T), jnp.float32),
            pltpu.VMEM((R, T), jnp.float32),
            pltpu.VMEM((R, T), jnp.float32),
            pltpu.SemaphoreType.DMA,
            pltpu.SemaphoreType.DMA,
            pltpu.SemaphoreType.DMA,
            pltpu.SemaphoreType.DMA,
            pltpu.SemaphoreType.DMA,
            pltpu.SemaphoreType.DMA,
        ],
        interpret=interpret,
        compiler_params=pltpu.CompilerParams(needs_layout_passes=False),
    )


def _warp_indices():
    wkey = jax.random.fold_in(jax.random.key(0), 1)
    warp = jnp.cumsum(jax.random.normal(wkey, (T,), dtype=jnp.float32) * SIGMA)
    warp = (warp - warp.min()) / (warp.max() - warp.min()) * (T - 1)
    return jnp.clip(warp.astype(jnp.int32), 0, T - 1)


@jax.jit
def kernel(x):
    idx = _warp_indices()
    out = _make_kernel()(x.reshape(ROWS, T), idx)
    return out.reshape(x.shape)
